# baseline (speedup 1.0000x reference)
import dataclasses
import functools

import jax
import jax.numpy as jnp
from jax import lax
from jax.experimental import pallas as pl
from jax.experimental.pallas import tpu as pltpu
from jax.experimental.pallas import tpu_sc as plsc

N = 10000
E = 160000
IN = 128
OUT = 16
DE = 4

NC = 2
NS = 16
NW = NC * NS
EPAD = 163840
EPT = EPAD // NW
C = 128
NCHUNK = EPT // C
NPAD = 10240
ROWS_PT = NPAD // NS


def _pre_body(v_ref, wu_ref, wrs_ref, brs_ref, u_ref, rs_ref):
    x = v_ref[...]
    dn = (((1,), (0,)), ((), ()))
    u_ref[...] = lax.dot_general(
        x, wu_ref[...], dn, precision=lax.Precision.HIGHEST,
        preferred_element_type=jnp.float32).astype(jnp.bfloat16)
    rs_ref[...] = lax.dot_general(
        x, wrs_ref[...], dn, precision=lax.Precision.HIGHEST,
        preferred_element_type=jnp.float32) + brs_ref[...]


def _sc_body(u_hbm, src_hbm, dst_hbm, ep_hbm, zero_hbm, out_hbm,
             src_all, dst_all, ce_all, rows0, rows1, rows2, rows3,
             msg0, msg1, msg2, msg3, agg_sh, u_sh,
             gsem0, gsem1, gsem2, gsem3, ssem0, ssem1, ssem2, ssem3):
    cid = lax.axis_index("c")
    sid = lax.axis_index("s")
    wid = cid * NS + sid

    r0 = sid * ROWS_PT
    pltpu.sync_copy(zero_hbm.at[pl.ds(r0, ROWS_PT)], agg_sh.at[pl.ds(r0, ROWS_PT)])
    pltpu.sync_copy(u_hbm.at[pl.ds(r0, ROWS_PT)], u_sh.at[pl.ds(r0, ROWS_PT)])

    pltpu.sync_copy(src_hbm.at[pl.ds(wid * NCHUNK, NCHUNK)], src_all)
    pltpu.sync_copy(dst_hbm.at[pl.ds(wid * NCHUNK, NCHUNK)], dst_all)
    pltpu.sync_copy(ep_hbm.at[:, pl.ds(wid * EPT, EPT)], ce_all)
    plsc.subcore_barrier()

    def gather_start(j, rows_ref, sem, from_hbm=False):
        table = u_hbm if from_hbm else u_sh
        pltpu.async_copy(table.at[src_all.at[j]], rows_ref, sem)

    def gather_wait(rows_ref, sem):
        pltpu.make_async_copy(u_sh.at[src_all.at[0]], rows_ref, sem).wait()

    def scatter_start(j, msg_ref, sem):
        pltpu.async_copy(msg_ref, agg_sh.at[dst_all.at[j]], sem, add=True)

    def scatter_wait(msg_ref, sem):
        pltpu.make_async_copy(msg_ref, agg_sh.at[dst_all.at[0]], sem).wait()

    def compute(j, rows_ref, msg_ref):
        goff = j * C

        @pl.loop(0, C // 16)
        def _group(g):
            cvs = [ce_all[d, pl.ds(goff + g * 16, 16)] for d in range(DE)]
            for ii in range(16):
                i = g * 16 + ii
                s0, s1 = plsc.unpack(rows_ref[i, pl.ds(0, 2 * OUT)],
                                     format=plsc.PackFormat.INTERLEAVED)
                s2, s3 = plsc.unpack(rows_ref[i, pl.ds(2 * OUT, 2 * OUT)],
                                     format=plsc.PackFormat.INTERLEAVED)
                s4, _unused = plsc.unpack(rows_ref[i, pl.ds(4 * OUT, 2 * OUT)],
                                          format=plsc.PackFormat.INTERLEAVED)
                segs = (s0, s1, s2, s3)
                acc = s4
                for d in range(DE):
                    acc = acc + _bcast(cvs[d], ii) * segs[d]
                msg_ref[i, :] = acc

    rows_bufs = [rows0, rows1, rows2, rows3]
    msg_bufs = [msg0, msg1, msg2, msg3]
    gsems = [gsem0, gsem1, gsem2, gsem3]
    ssems = [ssem0, ssem1, ssem2, ssem3]

    gather_start(0, rows0, gsem0)
    gather_start(1, rows1, gsem1, from_hbm=True)
    gather_start(2, rows2, gsem2)

    @pl.loop(0, NCHUNK, step=4)
    def _quad(j):
        for b in range(4):
            jb = j + b
            nb = (b + 3) % 4

            @pl.when(jb + 3 < NCHUNK)
            def _():
                gather_start(jb + 3, rows_bufs[nb], gsems[nb],
                             from_hbm=(nb % 2 == 1))

            gather_wait(rows_bufs[b], gsems[b])

            @pl.when(j > 0)
            def _():
                scatter_wait(msg_bufs[b], ssems[b])

            compute(jb, rows_bufs[b], msg_bufs[b])
            scatter_start(jb, msg_bufs[b], ssems[b])

    for b in range(4):
        scatter_wait(msg_bufs[b], ssems[b])

    plsc.subcore_barrier()
    pltpu.sync_copy(agg_sh.at[pl.ds(r0, ROWS_PT)],
                    out_hbm.at[cid, pl.ds(r0, ROWS_PT)])


_GDN = lax.GatherDimensionNumbers(
    offset_dims=(), collapsed_slice_dims=(0,), start_index_map=(0,))


def _bcast(vec, ii):
    idx = jnp.full((16, 1), ii, jnp.int32)
    return lax.gather(vec, idx, _GDN, (1,),
                      mode=lax.GatherScatterMode.PROMISE_IN_BOUNDS)


_sc_mesh = plsc.VectorSubcoreMesh(core_axis_name="c", subcore_axis_name="s")

_sc_cp = pltpu.CompilerParams(use_tc_tiling_on_sc=False)
if "needs_layout_passes" in pltpu.CompilerParams.__dataclass_fields__:
    _sc_cp = dataclasses.replace(_sc_cp, needs_layout_passes=False)

_sc_scatter = pl.kernel(
    _sc_body,
    compiler_params=_sc_cp,
    out_type=jax.ShapeDtypeStruct((NC, NPAD, OUT), jnp.float32),
    mesh=_sc_mesh,
    scratch_types=[
        pltpu.VMEM((NCHUNK, C), jnp.int32),
        pltpu.VMEM((NCHUNK, C), jnp.int32),
        pltpu.VMEM((DE + 1, EPT), jnp.float32),
        pltpu.VMEM((C, 6 * OUT), jnp.bfloat16),
        pltpu.VMEM((C, 6 * OUT), jnp.bfloat16),
        pltpu.VMEM((C, 6 * OUT), jnp.bfloat16),
        pltpu.VMEM((C, 6 * OUT), jnp.bfloat16),
        pltpu.VMEM((C, OUT), jnp.float32),
        pltpu.VMEM((C, OUT), jnp.float32),
        pltpu.VMEM((C, OUT), jnp.float32),
        pltpu.VMEM((C, OUT), jnp.float32),
        pltpu.VMEM_SHARED((NPAD, OUT), jnp.float32),
        pltpu.VMEM_SHARED((NPAD, 6 * OUT), jnp.bfloat16),
        pltpu.SemaphoreType.DMA,
        pltpu.SemaphoreType.DMA,
        pltpu.SemaphoreType.DMA,
        pltpu.SemaphoreType.DMA,
        pltpu.SemaphoreType.DMA,
        pltpu.SemaphoreType.DMA,
        pltpu.SemaphoreType.DMA,
        pltpu.SemaphoreType.DMA,
    ],
)


def _post_body(part_ref, rs_ref, g_ref, b_ref, o_ref):
    pre = part_ref[0, :N] + part_ref[1, :N] + rs_ref[:N, :OUT]
    mean = jnp.mean(pre, axis=0, keepdims=True)
    cen = pre - mean
    var = jnp.mean(cen * cen, axis=0, keepdims=True)
    xn = cen * lax.rsqrt(var + 1e-5) * g_ref[...] + b_ref[...]
    act = jnp.where(xn >= 0, xn, 0.01 * xn)
    o_ref[...] = act + rs_ref[:N, OUT:2 * OUT]


def kernel(v, e, edge_index, enet_W, enet_b, lin_root_W, conv_bias,
           bn_gamma, bn_beta, proj_W):
    src = edge_index[0]
    dst = edge_index[1]
    wu5 = jnp.concatenate(
        [enet_W.reshape(DE, IN, OUT).transpose(1, 0, 2),
         enet_b.reshape(IN, 1, OUT)], axis=1)
    zseg = jnp.zeros((IN, OUT), jnp.float32)
    wu = jnp.concatenate(
        [jnp.stack([wu5[:, 0], wu5[:, 1]], axis=2).reshape(IN, 2 * OUT),
         jnp.stack([wu5[:, 2], wu5[:, 3]], axis=2).reshape(IN, 2 * OUT),
         jnp.stack([wu5[:, 4], zseg], axis=2).reshape(IN, 2 * OUT)],
        axis=1)
    wrs = jnp.concatenate([lin_root_W, proj_W], axis=1)
    brs = jnp.concatenate(
        [conv_bias, jnp.zeros((OUT,), jnp.float32)]).reshape(1, 2 * OUT)

    pad = EPAD - E
    src_p = jnp.concatenate([src, jnp.full((pad,), N, jnp.int32)]).reshape(
        EPAD // C, C)
    dst_p = jnp.concatenate([dst, jnp.zeros((pad,), jnp.int32)]).reshape(
        EPAD // C, C)
    ep_t = jnp.concatenate(
        [jnp.concatenate([e.T, jnp.ones((1, E), jnp.float32)], axis=0),
         jnp.zeros((DE + 1, pad), jnp.float32)], axis=1)
    zero = jnp.zeros((NPAD, OUT), jnp.float32)

    v_p = jnp.concatenate([v, jnp.zeros((NPAD - N, IN), jnp.float32)], axis=0)
    u, rs = pl.pallas_call(
        _pre_body,
        out_shape=[jax.ShapeDtypeStruct((NPAD, 6 * OUT), jnp.bfloat16),
                   jax.ShapeDtypeStruct((NPAD, 2 * OUT), jnp.float32)],
    )(v_p, wu, wrs, brs)

    parts = _sc_scatter(u, src_p, dst_p, ep_t, zero)

    out = pl.pallas_call(
        _post_body,
        out_shape=jax.ShapeDtypeStruct((N, OUT), jnp.float32),
    )(parts, rs, bn_gamma.reshape(1, OUT), bn_beta.reshape(1, OUT))
    return out


# speedup vs baseline: 1.0138x; 1.0138x over previous
import dataclasses
import functools

import jax
import jax.numpy as jnp
from jax import lax
from jax.experimental import pallas as pl
from jax.experimental.pallas import tpu as pltpu
from jax.experimental.pallas import tpu_sc as plsc

N = 10000
E = 160000
IN = 128
OUT = 16
DE = 4

NC = 2
NS = 16
NW = NC * NS
EPAD = 163840
EPT = EPAD // NW
C = 128
NCHUNK = EPT // C
NPAD = 10240
ROWS_PT = NPAD // NS


def _pre_body(v_ref, wu_ref, wrs_ref, brs_ref, u_ref, rs_ref):
    x = v_ref[...]
    dn = (((1,), (0,)), ((), ()))
    u_ref[...] = lax.dot_general(
        x, wu_ref[...], dn, precision=lax.Precision.HIGHEST,
        preferred_element_type=jnp.float32).astype(jnp.bfloat16)
    rs_ref[...] = lax.dot_general(
        x, wrs_ref[...], dn, precision=lax.Precision.HIGHEST,
        preferred_element_type=jnp.float32) + brs_ref[...]


def _sc_body(u_hbm, src_hbm, dst_hbm, ep_hbm, zero_hbm, out_hbm,
             src_all, dst_all, ce_all, rows0, rows1, rows2, rows3,
             msg0, msg1, msg2, msg3, agg_sh, u_sh,
             gsem0, gsem1, gsem2, gsem3, ssem0, ssem1, ssem2, ssem3):
    cid = lax.axis_index("c")
    sid = lax.axis_index("s")
    wid = cid * NS + sid

    r0 = sid * ROWS_PT
    pltpu.sync_copy(zero_hbm.at[pl.ds(r0, ROWS_PT)], agg_sh.at[pl.ds(r0, ROWS_PT)])
    pltpu.sync_copy(u_hbm.at[pl.ds(r0, ROWS_PT)], u_sh.at[pl.ds(r0, ROWS_PT)])

    pltpu.sync_copy(src_hbm.at[pl.ds(wid * NCHUNK, NCHUNK)], src_all)
    pltpu.sync_copy(dst_hbm.at[pl.ds(wid * NCHUNK, NCHUNK)], dst_all)
    pltpu.sync_copy(ep_hbm.at[:, pl.ds(wid * EPT, EPT)], ce_all)
    plsc.subcore_barrier()

    def gather_start(j, rows_ref, sem):
        pltpu.async_copy(u_sh.at[src_all.at[j]], rows_ref, sem)

    def gather_wait(rows_ref, sem):
        pltpu.make_async_copy(u_sh.at[src_all.at[0]], rows_ref, sem).wait()

    def scatter_start(j, msg_ref, sem):
        pltpu.async_copy(msg_ref, agg_sh.at[dst_all.at[j]], sem, add=True)

    def scatter_wait(msg_ref, sem):
        pltpu.make_async_copy(msg_ref, agg_sh.at[dst_all.at[0]], sem).wait()

    def compute(j, rows_ref, msg_ref):
        goff = j * C

        @pl.loop(0, C, unroll=4)
        def _edge(i):
            s0, s1 = plsc.unpack(rows_ref[i, pl.ds(0, 2 * OUT)],
                                 format=plsc.PackFormat.INTERLEAVED)
            s2, s3 = plsc.unpack(rows_ref[i, pl.ds(2 * OUT, 2 * OUT)],
                                 format=plsc.PackFormat.INTERLEAVED)
            s4, _unused = plsc.unpack(rows_ref[i, pl.ds(4 * OUT, 2 * OUT)],
                                      format=plsc.PackFormat.INTERLEAVED)
            segs = (s0, s1, s2, s3)
            acc = s4
            for d in range(DE):
                cvec = plsc.load_gather(
                    ce_all, [jnp.full((16,), d, jnp.int32),
                             jnp.full((16,), goff + i, jnp.int32)])
                acc = acc + cvec * segs[d]
            msg_ref[i, :] = acc

    rows_bufs = [rows0, rows1, rows2, rows3]
    msg_bufs = [msg0, msg1, msg2, msg3]
    gsems = [gsem0, gsem1, gsem2, gsem3]
    ssems = [ssem0, ssem1, ssem2, ssem3]

    gather_start(0, rows0, gsem0)
    gather_start(1, rows1, gsem1)
    gather_start(2, rows2, gsem2)

    @pl.loop(0, NCHUNK, step=4)
    def _quad(j):
        for b in range(4):
            jb = j + b
            nb = (b + 3) % 4

            @pl.when(jb + 3 < NCHUNK)
            def _():
                gather_start(jb + 3, rows_bufs[nb], gsems[nb])

            gather_wait(rows_bufs[b], gsems[b])

            @pl.when(j > 0)
            def _():
                scatter_wait(msg_bufs[b], ssems[b])

            compute(jb, rows_bufs[b], msg_bufs[b])
            scatter_start(jb, msg_bufs[b], ssems[b])

    for b in range(4):
        scatter_wait(msg_bufs[b], ssems[b])

    plsc.subcore_barrier()
    pltpu.sync_copy(agg_sh.at[pl.ds(r0, ROWS_PT)],
                    out_hbm.at[cid, pl.ds(r0, ROWS_PT)])


_sc_mesh = plsc.VectorSubcoreMesh(core_axis_name="c", subcore_axis_name="s")

_sc_cp = pltpu.CompilerParams(use_tc_tiling_on_sc=False)
if "needs_layout_passes" in pltpu.CompilerParams.__dataclass_fields__:
    _sc_cp = dataclasses.replace(_sc_cp, needs_layout_passes=False)

_sc_scatter = pl.kernel(
    _sc_body,
    compiler_params=_sc_cp,
    out_type=jax.ShapeDtypeStruct((NC, NPAD, OUT), jnp.float32),
    mesh=_sc_mesh,
    scratch_types=[
        pltpu.VMEM((NCHUNK, C), jnp.int32),
        pltpu.VMEM((NCHUNK, C), jnp.int32),
        pltpu.VMEM((DE + 1, EPT), jnp.float32),
        pltpu.VMEM((C, 6 * OUT), jnp.bfloat16),
        pltpu.VMEM((C, 6 * OUT), jnp.bfloat16),
        pltpu.VMEM((C, 6 * OUT), jnp.bfloat16),
        pltpu.VMEM((C, 6 * OUT), jnp.bfloat16),
        pltpu.VMEM((C, OUT), jnp.float32),
        pltpu.VMEM((C, OUT), jnp.float32),
        pltpu.VMEM((C, OUT), jnp.float32),
        pltpu.VMEM((C, OUT), jnp.float32),
        pltpu.VMEM_SHARED((NPAD, OUT), jnp.float32),
        pltpu.VMEM_SHARED((NPAD, 6 * OUT), jnp.bfloat16),
        pltpu.SemaphoreType.DMA,
        pltpu.SemaphoreType.DMA,
        pltpu.SemaphoreType.DMA,
        pltpu.SemaphoreType.DMA,
        pltpu.SemaphoreType.DMA,
        pltpu.SemaphoreType.DMA,
        pltpu.SemaphoreType.DMA,
        pltpu.SemaphoreType.DMA,
    ],
)


def _post_body(part_ref, rs_ref, g_ref, b_ref, o_ref):
    pre = part_ref[0, :N] + part_ref[1, :N] + rs_ref[:N, :OUT]
    mean = jnp.mean(pre, axis=0, keepdims=True)
    cen = pre - mean
    var = jnp.mean(cen * cen, axis=0, keepdims=True)
    xn = cen * lax.rsqrt(var + 1e-5) * g_ref[...] + b_ref[...]
    act = jnp.where(xn >= 0, xn, 0.01 * xn)
    o_ref[...] = act + rs_ref[:N, OUT:2 * OUT]


def kernel(v, e, edge_index, enet_W, enet_b, lin_root_W, conv_bias,
           bn_gamma, bn_beta, proj_W):
    src = edge_index[0]
    dst = edge_index[1]
    wu5 = jnp.concatenate(
        [enet_W.reshape(DE, IN, OUT).transpose(1, 0, 2),
         enet_b.reshape(IN, 1, OUT)], axis=1)
    zseg = jnp.zeros((IN, OUT), jnp.float32)
    wu = jnp.concatenate(
        [jnp.stack([wu5[:, 0], wu5[:, 1]], axis=2).reshape(IN, 2 * OUT),
         jnp.stack([wu5[:, 2], wu5[:, 3]], axis=2).reshape(IN, 2 * OUT),
         jnp.stack([wu5[:, 4], zseg], axis=2).reshape(IN, 2 * OUT)],
        axis=1)
    wrs = jnp.concatenate([lin_root_W, proj_W], axis=1)
    brs = jnp.concatenate(
        [conv_bias, jnp.zeros((OUT,), jnp.float32)]).reshape(1, 2 * OUT)

    pad = EPAD - E
    src_p = jnp.concatenate([src, jnp.full((pad,), N, jnp.int32)]).reshape(
        EPAD // C, C)
    dst_p = jnp.concatenate([dst, jnp.zeros((pad,), jnp.int32)]).reshape(
        EPAD // C, C)
    ep_t = jnp.concatenate(
        [jnp.concatenate([e.T, jnp.ones((1, E), jnp.float32)], axis=0),
         jnp.zeros((DE + 1, pad), jnp.float32)], axis=1)
    zero = jnp.zeros((NPAD, OUT), jnp.float32)

    v_p = jnp.concatenate([v, jnp.zeros((NPAD - N, IN), jnp.float32)], axis=0)
    u, rs = pl.pallas_call(
        _pre_body,
        out_shape=[jax.ShapeDtypeStruct((NPAD, 6 * OUT), jnp.bfloat16),
                   jax.ShapeDtypeStruct((NPAD, 2 * OUT), jnp.float32)],
    )(v_p, wu, wrs, brs)

    parts = _sc_scatter(u, src_p, dst_p, ep_t, zero)

    out = pl.pallas_call(
        _post_body,
        out_shape=jax.ShapeDtypeStruct((N, OUT), jnp.float32),
    )(parts, rs, bn_gamma.reshape(1, OUT), bn_beta.reshape(1, OUT))
    return out
